# baseline (speedup 1.0000x reference)
import functools

import jax
import jax.numpy as jnp
from jax import lax
from jax.experimental import pallas as pl
from jax.experimental.pallas import tpu as pltpu
from jax.experimental.pallas import tpu_sc as plsc

NC = 2
NS = 16
NW = NC * NS
CHUNK = 128
G = 32


def _flt(x):
    return x.astype(jnp.float32)


@functools.lru_cache(maxsize=None)
def _make_deg_kernel(n_pad, d, cpw):
    rows_per_sub = n_pad // NS
    nz_full, nz_rem = divmod(rows_per_sub, CHUNK)
    mesh = plsc.VectorSubcoreMesh(core_axis_name="c", subcore_axis_name="s")

    @functools.partial(
        pl.kernel,
        out_type=jax.ShapeDtypeStruct((NC, n_pad, d), jnp.float32),
        mesh=mesh,
        scratch_types=[
            pltpu.VMEM((cpw, CHUNK), jnp.int32),
            pltpu.VMEM((CHUNK, d), jnp.float32),
            pltpu.VMEM_SHARED((n_pad, d), jnp.float32),
        ],
    )
    def deg_kernel(dstc_hbm, out_hbm, dst_v, buf_v, deg_sh):
        c = lax.axis_index("c")
        s = lax.axis_index("s")
        wid = c * NS + s
        pltpu.sync_copy(dstc_hbm.at[pl.ds(wid * cpw, cpw)], dst_v)

        zero16 = jnp.zeros((16,), jnp.float32)

        def fill_zeros(i, _):
            for k in range(d // 16):
                buf_v[i, pl.ds(k * 16, 16)] = zero16
            return 0

        lax.fori_loop(0, CHUNK, fill_zeros, 0)
        base = s * rows_per_sub
        for k in range(nz_full):
            pltpu.sync_copy(buf_v, deg_sh.at[pl.ds(base + k * CHUNK, CHUNK)])
        if nz_rem:
            pltpu.sync_copy(buf_v.at[pl.ds(0, nz_rem)],
                            deg_sh.at[pl.ds(base + nz_full * CHUNK, nz_rem)])
        plsc.subcore_barrier()

        one16 = jnp.ones((16,), jnp.float32)

        def fill_ones(i, _):
            for k in range(d // 16):
                buf_v[i, pl.ds(k * 16, 16)] = one16
            return 0

        lax.fori_loop(0, CHUNK, fill_ones, 0)

        def scat(j, _):
            pltpu.sync_copy(buf_v, deg_sh.at[dst_v.at[j]], add=True)
            return 0

        lax.fori_loop(0, cpw, scat, 0)
        plsc.subcore_barrier()

        for k in range(nz_full):
            pltpu.sync_copy(deg_sh.at[pl.ds(base + k * CHUNK, CHUNK)], buf_v)
            pltpu.sync_copy(buf_v, out_hbm.at[c, pl.ds(base + k * CHUNK, CHUNK)])
        if nz_rem:
            pltpu.sync_copy(deg_sh.at[pl.ds(base + nz_full * CHUNK, nz_rem)],
                            buf_v.at[pl.ds(0, nz_rem)])
            pltpu.sync_copy(buf_v.at[pl.ds(0, nz_rem)],
                            out_hbm.at[c, pl.ds(base + nz_full * CHUNK, nz_rem)])

    return deg_kernel


@functools.lru_cache(maxsize=None)
def _make_seg_kernel(n_pad, d, a_chunks, b_chunks):
    rows_per_sub = n_pad // NS
    nz_full, nz_rem = divmod(rows_per_sub, CHUNK)
    ngroups_max = max(a_chunks, b_chunks) // G
    mesh = plsc.VectorSubcoreMesh(core_axis_name="c", subcore_axis_name="s")

    @functools.partial(
        pl.kernel,
        out_type=jax.ShapeDtypeStruct((n_pad, d), jnp.float32),
        mesh=mesh,
        scratch_types=[
            pltpu.VMEM((G, CHUNK), jnp.int32),
            pltpu.VMEM((G, CHUNK), jnp.int32),
            pltpu.VMEM((CHUNK, d), jnp.float32),
            pltpu.VMEM((CHUNK, d), jnp.float32),
            pltpu.VMEM_SHARED((n_pad, d), jnp.float32),
            pltpu.SemaphoreType.DMA,
            pltpu.SemaphoreType.DMA,
            pltpu.SemaphoreType.DMA,
            pltpu.SemaphoreType.DMA,
        ],
    )
    def seg_kernel(m_hbm, srcc_hbm, dstc_hbm, out_hbm,
                   src_g, dst_g, rows0, rows1, agg_sh, gs0, gs1, ss0, ss1):
        c = lax.axis_index("c")
        s = lax.axis_index("s")
        e0 = s * a_chunks
        ngroups = jnp.where(c == 0, a_chunks // G, b_chunks // G)

        zero16 = jnp.zeros((16,), jnp.float32)

        def fill_zeros(i, _):
            for k in range(d // 16):
                rows0[i, pl.ds(k * 16, 16)] = zero16
            return 0

        with jax.named_scope("segzero"):
            @pl.when(c == 0)
            def _():
                lax.fori_loop(0, CHUNK, fill_zeros, 0)
                zbase = s * rows_per_sub
                for k in range(nz_full):
                    pltpu.sync_copy(rows0,
                                    agg_sh.at[pl.ds(zbase + k * CHUNK, CHUNK)])
                if nz_rem:
                    pltpu.sync_copy(rows0.at[pl.ds(0, nz_rem)],
                                    agg_sh.at[pl.ds(zbase + nz_full * CHUNK,
                                                    nz_rem)])
            plsc.subcore_barrier()

        def group(g, _):
            @pl.when(g < ngroups)
            def _():
                gbase = e0 + g * G
                pltpu.sync_copy(srcc_hbm.at[pl.ds(gbase, G)], src_g)
                pltpu.sync_copy(dstc_hbm.at[pl.ds(gbase, G)], dst_g)
                pltpu.async_copy(m_hbm.at[src_g.at[0]], rows0, gs0)

                def pair(t, _):
                    a = 2 * t
                    pltpu.make_async_copy(m_hbm.at[src_g.at[a]], rows0,
                                          gs0).wait()

                    @pl.when((g > 0) | (t > 0))
                    def _():
                        pltpu.make_async_copy(rows1, agg_sh.at[dst_g.at[a]],
                                              ss1).wait()

                    pltpu.async_copy(m_hbm.at[src_g.at[a + 1]], rows1, gs1)
                    pltpu.async_copy(rows0, agg_sh.at[dst_g.at[a]], ss0,
                                     add=True)
                    pltpu.make_async_copy(m_hbm.at[src_g.at[a + 1]], rows1,
                                          gs1).wait()
                    pltpu.make_async_copy(rows0, agg_sh.at[dst_g.at[a]],
                                          ss0).wait()

                    @pl.when(a + 2 < G)
                    def _():
                        pltpu.async_copy(m_hbm.at[src_g.at[a + 2]], rows0, gs0)

                    pltpu.async_copy(rows1, agg_sh.at[dst_g.at[a + 1]], ss1,
                                     add=True)
                    return 0

                lax.fori_loop(0, G // 2, pair, 0)
            return 0

        with jax.named_scope("segedges"):
            lax.fori_loop(0, ngroups_max, group, 0)
            @pl.when((c == 0) & (ngroups > 0))
            def _():
                pltpu.make_async_copy(rows1, agg_sh.at[dst_g.at[G - 1]],
                                      ss1).wait()
            plsc.subcore_barrier()

        with jax.named_scope("segcopyout"):
            @pl.when(c == 0)
            def _():
                zbase = s * rows_per_sub
                for k in range(nz_full):
                    pltpu.sync_copy(agg_sh.at[pl.ds(zbase + k * CHUNK, CHUNK)],
                                    rows0)
                    pltpu.sync_copy(rows0,
                                    out_hbm.at[pl.ds(zbase + k * CHUNK, CHUNK)])
                if nz_rem:
                    pltpu.sync_copy(agg_sh.at[pl.ds(zbase + nz_full * CHUNK,
                                                    nz_rem)],
                                    rows0.at[pl.ds(0, nz_rem)])
                    pltpu.sync_copy(rows0.at[pl.ds(0, nz_rem)],
                                    out_hbm.at[pl.ds(zbase + nz_full * CHUNK,
                                                     nz_rem)])

    return seg_kernel


def _tc_first(x, w_self, w_neigh, b, block):
    n, d = x.shape
    h = w_self.shape[1]

    def body(x_ref, ws_ref, wn_ref, b_ref, s_ref, m_ref):
        xb = x_ref[...]
        s_ref[...] = jnp.dot(xb, ws_ref[...],
                             preferred_element_type=jnp.float32) + b_ref[...]
        m_ref[...] = jnp.dot(xb, wn_ref[...], preferred_element_type=jnp.float32)

    return pl.pallas_call(
        body,
        grid=(n // block,),
        in_specs=[
            pl.BlockSpec((block, d), lambda i: (i, 0)),
            pl.BlockSpec((d, h), lambda i: (0, 0)),
            pl.BlockSpec((d, h), lambda i: (0, 0)),
            pl.BlockSpec((1, h), lambda i: (0, 0)),
        ],
        out_specs=[
            pl.BlockSpec((block, h), lambda i: (i, 0)),
            pl.BlockSpec((block, h), lambda i: (i, 0)),
        ],
        out_shape=[
            jax.ShapeDtypeStruct((n, h), jnp.float32),
            jax.ShapeDtypeStruct((n, h), jnp.float32),
        ],
    )(x, w_self, w_neigh, b.reshape(1, h))


def _tc_mid(s_prev, p0, dg0, dg1, w_self, w_neigh, b, block):
    n, d = s_prev.shape
    h = w_self.shape[1]

    def body(s_ref, p0_ref, dg0_ref, dg1_ref, ws_ref, wn_ref, b_ref,
             so_ref, mo_ref):
        inv = 1.0 / jnp.maximum(dg0_ref[...] + dg1_ref[...], 1.0)
        hb = jnp.maximum(s_ref[...] + p0_ref[...] * inv, 0.0)
        so_ref[...] = jnp.dot(hb, ws_ref[...],
                              preferred_element_type=jnp.float32) + b_ref[...]
        mo_ref[...] = jnp.dot(hb, wn_ref[...], preferred_element_type=jnp.float32)

    return pl.pallas_call(
        body,
        grid=(n // block,),
        in_specs=[
            pl.BlockSpec((block, d), lambda i: (i, 0)),
            pl.BlockSpec((block, d), lambda i: (i, 0)),
            pl.BlockSpec((block, 1), lambda i: (i, 0)),
            pl.BlockSpec((block, 1), lambda i: (i, 0)),
            pl.BlockSpec((d, h), lambda i: (0, 0)),
            pl.BlockSpec((d, h), lambda i: (0, 0)),
            pl.BlockSpec((1, h), lambda i: (0, 0)),
        ],
        out_specs=[
            pl.BlockSpec((block, h), lambda i: (i, 0)),
            pl.BlockSpec((block, h), lambda i: (i, 0)),
        ],
        out_shape=[
            jax.ShapeDtypeStruct((n, h), jnp.float32),
            jax.ShapeDtypeStruct((n, h), jnp.float32),
        ],
    )(s_prev, p0, dg0, dg1, w_self, w_neigh, b.reshape(1, h))


def _tc_last(s_prev, p0, dg0, dg1, block):
    n, d = s_prev.shape

    def body(s_ref, p0_ref, dg0_ref, dg1_ref, o_ref):
        inv = 1.0 / jnp.maximum(dg0_ref[...] + dg1_ref[...], 1.0)
        o_ref[...] = s_ref[...] + p0_ref[...] * inv

    return pl.pallas_call(
        body,
        grid=(n // block,),
        in_specs=[
            pl.BlockSpec((block, d), lambda i: (i, 0)),
            pl.BlockSpec((block, d), lambda i: (i, 0)),
            pl.BlockSpec((block, 1), lambda i: (i, 0)),
            pl.BlockSpec((block, 1), lambda i: (i, 0)),
        ],
        out_specs=pl.BlockSpec((block, d), lambda i: (i, 0)),
        out_shape=jax.ShapeDtypeStruct((n, d), jnp.float32),
    )(s_prev, p0, dg0, dg1)


def kernel(x, edge_index, W_self0, W_neigh0, b0, W_self1, W_neigh1, b1,
           W_self2, W_neigh2, b2):
    n, d = x.shape
    e = edge_index.shape[1]
    block = 1000 if n % 1000 == 0 else 8 * (n // 8)

    chunks_tot = -(-e // (CHUNK * NS * G)) * NS * G
    e_pad = chunks_tot * CHUNK
    cpw = chunks_tot // NW
    a_chunks = chunks_tot // NS
    b_chunks = 0
    span = -(-(n + 1) // (NS * 8)) * 8
    n_pad = span * NS

    src = edge_index[0]
    dst = edge_index[1]
    trash = n + jnp.arange(e_pad - e, dtype=jnp.int32) % (n_pad - n)
    srcc = jnp.concatenate(
        [src, jnp.zeros((e_pad - e,), jnp.int32)]).reshape(chunks_tot, CHUNK)
    dstc = jnp.concatenate([dst, trash]).reshape(chunks_tot, CHUNK)

    deg2 = _make_deg_kernel(n_pad, d, cpw)(dstc)
    dg0 = deg2[0, :n, 0:1]
    dg1 = deg2[1, :n, 0:1]

    seg = _make_seg_kernel(n_pad, d, a_chunks, b_chunks)

    s0, m0 = _tc_first(_flt(x), _flt(W_self0), _flt(W_neigh0), _flt(b0), block)
    p = seg(m0, srcc, dstc)
    s1, m1 = _tc_mid(s0, p[:n], dg0, dg1,
                     _flt(W_self1), _flt(W_neigh1), _flt(b1), block)
    p = seg(m1, srcc, dstc)
    s2, m2 = _tc_mid(s1, p[:n], dg0, dg1,
                     _flt(W_self2), _flt(W_neigh2), _flt(b2), block)
    p = seg(m2, srcc, dstc)
    return _tc_last(s2, p[:n], dg0, dg1, block)


# speedup vs baseline: 1.1083x; 1.1083x over previous
import functools

import jax
import jax.numpy as jnp
from jax import lax
from jax.experimental import pallas as pl
from jax.experimental.pallas import tpu as pltpu
from jax.experimental.pallas import tpu_sc as plsc

NC = 2
NS = 16
NW = NC * NS
CHUNK = 128
G = 32


def _flt(x):
    return x.astype(jnp.float32)


@functools.lru_cache(maxsize=None)
def _make_deg_kernel(n_pad, d, cpw):
    rows_per_sub = n_pad // NS
    nz_full, nz_rem = divmod(rows_per_sub, CHUNK)
    mesh = plsc.VectorSubcoreMesh(core_axis_name="c", subcore_axis_name="s")

    @functools.partial(
        pl.kernel,
        out_type=jax.ShapeDtypeStruct((NC, n_pad, d), jnp.float32),
        mesh=mesh,
        scratch_types=[
            pltpu.VMEM((cpw, CHUNK), jnp.int32),
            pltpu.VMEM((CHUNK, d), jnp.float32),
            pltpu.VMEM_SHARED((n_pad, d), jnp.float32),
        ],
    )
    def deg_kernel(dstc_hbm, out_hbm, dst_v, buf_v, deg_sh):
        c = lax.axis_index("c")
        s = lax.axis_index("s")
        wid = c * NS + s
        pltpu.sync_copy(dstc_hbm.at[pl.ds(wid * cpw, cpw)], dst_v)

        zero16 = jnp.zeros((16,), jnp.float32)

        def fill_zeros(i, _):
            for k in range(d // 16):
                buf_v[i, pl.ds(k * 16, 16)] = zero16
            return 0

        lax.fori_loop(0, CHUNK, fill_zeros, 0)
        base = s * rows_per_sub
        for k in range(nz_full):
            pltpu.sync_copy(buf_v, deg_sh.at[pl.ds(base + k * CHUNK, CHUNK)])
        if nz_rem:
            pltpu.sync_copy(buf_v.at[pl.ds(0, nz_rem)],
                            deg_sh.at[pl.ds(base + nz_full * CHUNK, nz_rem)])
        plsc.subcore_barrier()

        one16 = jnp.ones((16,), jnp.float32)

        def fill_ones(i, _):
            for k in range(d // 16):
                buf_v[i, pl.ds(k * 16, 16)] = one16
            return 0

        lax.fori_loop(0, CHUNK, fill_ones, 0)

        def scat(j, _):
            pltpu.sync_copy(buf_v, deg_sh.at[dst_v.at[j]], add=True)
            return 0

        lax.fori_loop(0, cpw, scat, 0)
        plsc.subcore_barrier()

        for k in range(nz_full):
            pltpu.sync_copy(deg_sh.at[pl.ds(base + k * CHUNK, CHUNK)], buf_v)
            pltpu.sync_copy(buf_v, out_hbm.at[c, pl.ds(base + k * CHUNK, CHUNK)])
        if nz_rem:
            pltpu.sync_copy(deg_sh.at[pl.ds(base + nz_full * CHUNK, nz_rem)],
                            buf_v.at[pl.ds(0, nz_rem)])
            pltpu.sync_copy(buf_v.at[pl.ds(0, nz_rem)],
                            out_hbm.at[c, pl.ds(base + nz_full * CHUNK, nz_rem)])

    return deg_kernel


@functools.lru_cache(maxsize=None)
def _make_seg_kernel(n_pad, d, a_chunks, b_chunks):
    rows_per_sub = n_pad // NS
    nz_full, nz_rem = divmod(rows_per_sub, CHUNK)
    ngroups_max = max(a_chunks, b_chunks) // G
    mesh = plsc.VectorSubcoreMesh(core_axis_name="c", subcore_axis_name="s")

    @functools.partial(
        pl.kernel,
        out_type=jax.ShapeDtypeStruct((NC, n_pad, d), jnp.float32),
        mesh=mesh,
        scratch_types=[
            pltpu.VMEM((G, CHUNK), jnp.int32),
            pltpu.VMEM((G, CHUNK), jnp.int32),
            pltpu.VMEM((CHUNK, d), jnp.float32),
            pltpu.VMEM((CHUNK, d), jnp.float32),
            pltpu.VMEM_SHARED((n_pad, d), jnp.float32),
            pltpu.SemaphoreType.DMA,
            pltpu.SemaphoreType.DMA,
            pltpu.SemaphoreType.DMA,
            pltpu.SemaphoreType.DMA,
        ],
    )
    def seg_kernel(m_hbm, srcc_hbm, dstc_hbm, out_hbm,
                   src_g, dst_g, rows0, rows1, agg_sh, gs0, gs1, ss0, ss1):
        c = lax.axis_index("c")
        s = lax.axis_index("s")
        e0 = jnp.where(c == 0, s * a_chunks, NS * a_chunks + s * b_chunks)
        ngroups = jnp.where(c == 0, a_chunks // G, b_chunks // G)

        zero16 = jnp.zeros((16,), jnp.float32)

        def fill_zeros(i, _):
            for k in range(d // 16):
                rows0[i, pl.ds(k * 16, 16)] = zero16
            return 0

        with jax.named_scope("segzero"):
            lax.fori_loop(0, CHUNK, fill_zeros, 0)
            zbase = s * rows_per_sub
            for k in range(nz_full):
                pltpu.sync_copy(rows0,
                                agg_sh.at[pl.ds(zbase + k * CHUNK, CHUNK)])
            if nz_rem:
                pltpu.sync_copy(rows0.at[pl.ds(0, nz_rem)],
                                agg_sh.at[pl.ds(zbase + nz_full * CHUNK,
                                                nz_rem)])
            plsc.subcore_barrier()

        def group(g, _):
            @pl.when(g < ngroups)
            def _():
                gbase = e0 + g * G
                pltpu.sync_copy(srcc_hbm.at[pl.ds(gbase, G)], src_g)
                pltpu.sync_copy(dstc_hbm.at[pl.ds(gbase, G)], dst_g)
                pltpu.async_copy(m_hbm.at[src_g.at[0]], rows0, gs0)

                def pair(t, _):
                    a = 2 * t
                    pltpu.make_async_copy(m_hbm.at[src_g.at[a]], rows0,
                                          gs0).wait()

                    @pl.when((g > 0) | (t > 0))
                    def _():
                        pltpu.make_async_copy(rows1, agg_sh.at[dst_g.at[a]],
                                              ss1).wait()

                    pltpu.async_copy(m_hbm.at[src_g.at[a + 1]], rows1, gs1)
                    pltpu.async_copy(rows0, agg_sh.at[dst_g.at[a]], ss0,
                                     add=True)
                    pltpu.make_async_copy(m_hbm.at[src_g.at[a + 1]], rows1,
                                          gs1).wait()
                    pltpu.make_async_copy(rows0, agg_sh.at[dst_g.at[a]],
                                          ss0).wait()

                    @pl.when(a + 2 < G)
                    def _():
                        pltpu.async_copy(m_hbm.at[src_g.at[a + 2]], rows0, gs0)

                    pltpu.async_copy(rows1, agg_sh.at[dst_g.at[a + 1]], ss1,
                                     add=True)
                    return 0

                lax.fori_loop(0, G // 2, pair, 0)
            return 0

        with jax.named_scope("segedges"):
            lax.fori_loop(0, ngroups_max, group, 0)
            @pl.when(ngroups > 0)
            def _():
                pltpu.make_async_copy(rows1, agg_sh.at[dst_g.at[G - 1]],
                                      ss1).wait()
            plsc.subcore_barrier()

        with jax.named_scope("segcopyout"):
            for k in range(nz_full):
                pltpu.sync_copy(agg_sh.at[pl.ds(zbase + k * CHUNK, CHUNK)],
                                rows0)
                pltpu.sync_copy(rows0,
                                out_hbm.at[c, pl.ds(zbase + k * CHUNK, CHUNK)])
            if nz_rem:
                pltpu.sync_copy(agg_sh.at[pl.ds(zbase + nz_full * CHUNK,
                                                nz_rem)],
                                rows0.at[pl.ds(0, nz_rem)])
                pltpu.sync_copy(rows0.at[pl.ds(0, nz_rem)],
                                out_hbm.at[c, pl.ds(zbase + nz_full * CHUNK,
                                                    nz_rem)])

    return seg_kernel


def _tc_first(x, w_self, w_neigh, b, block):
    n, d = x.shape
    h = w_self.shape[1]

    def body(x_ref, ws_ref, wn_ref, b_ref, s_ref, m_ref):
        xb = x_ref[...]
        s_ref[...] = jnp.dot(xb, ws_ref[...],
                             preferred_element_type=jnp.float32) + b_ref[...]
        m_ref[...] = jnp.dot(xb, wn_ref[...], preferred_element_type=jnp.float32)

    return pl.pallas_call(
        body,
        grid=(n // block,),
        in_specs=[
            pl.BlockSpec((block, d), lambda i: (i, 0)),
            pl.BlockSpec((d, h), lambda i: (0, 0)),
            pl.BlockSpec((d, h), lambda i: (0, 0)),
            pl.BlockSpec((1, h), lambda i: (0, 0)),
        ],
        out_specs=[
            pl.BlockSpec((block, h), lambda i: (i, 0)),
            pl.BlockSpec((block, h), lambda i: (i, 0)),
        ],
        out_shape=[
            jax.ShapeDtypeStruct((n, h), jnp.float32),
            jax.ShapeDtypeStruct((n, h), jnp.float32),
        ],
    )(x, w_self, w_neigh, b.reshape(1, h))


def _tc_mid(s_prev, p0, p1, dg0, dg1, w_self, w_neigh, b, block):
    n, d = s_prev.shape
    h = w_self.shape[1]

    def body(s_ref, p0_ref, p1_ref, dg0_ref, dg1_ref, ws_ref, wn_ref, b_ref,
             so_ref, mo_ref):
        inv = 1.0 / jnp.maximum(dg0_ref[...] + dg1_ref[...], 1.0)
        hb = jnp.maximum(s_ref[...] + (p0_ref[...] + p1_ref[...]) * inv, 0.0)
        so_ref[...] = jnp.dot(hb, ws_ref[...],
                              preferred_element_type=jnp.float32) + b_ref[...]
        mo_ref[...] = jnp.dot(hb, wn_ref[...], preferred_element_type=jnp.float32)

    return pl.pallas_call(
        body,
        grid=(n // block,),
        in_specs=[
            pl.BlockSpec((block, d), lambda i: (i, 0)),
            pl.BlockSpec((block, d), lambda i: (i, 0)),
            pl.BlockSpec((block, d), lambda i: (i, 0)),
            pl.BlockSpec((block, 1), lambda i: (i, 0)),
            pl.BlockSpec((block, 1), lambda i: (i, 0)),
            pl.BlockSpec((d, h), lambda i: (0, 0)),
            pl.BlockSpec((d, h), lambda i: (0, 0)),
            pl.BlockSpec((1, h), lambda i: (0, 0)),
        ],
        out_specs=[
            pl.BlockSpec((block, h), lambda i: (i, 0)),
            pl.BlockSpec((block, h), lambda i: (i, 0)),
        ],
        out_shape=[
            jax.ShapeDtypeStruct((n, h), jnp.float32),
            jax.ShapeDtypeStruct((n, h), jnp.float32),
        ],
    )(s_prev, p0, p1, dg0, dg1, w_self, w_neigh, b.reshape(1, h))


def _tc_last(s_prev, p0, p1, dg0, dg1, block):
    n, d = s_prev.shape

    def body(s_ref, p0_ref, p1_ref, dg0_ref, dg1_ref, o_ref):
        inv = 1.0 / jnp.maximum(dg0_ref[...] + dg1_ref[...], 1.0)
        o_ref[...] = s_ref[...] + (p0_ref[...] + p1_ref[...]) * inv

    return pl.pallas_call(
        body,
        grid=(n // block,),
        in_specs=[
            pl.BlockSpec((block, d), lambda i: (i, 0)),
            pl.BlockSpec((block, d), lambda i: (i, 0)),
            pl.BlockSpec((block, d), lambda i: (i, 0)),
            pl.BlockSpec((block, 1), lambda i: (i, 0)),
            pl.BlockSpec((block, 1), lambda i: (i, 0)),
        ],
        out_specs=pl.BlockSpec((block, d), lambda i: (i, 0)),
        out_shape=jax.ShapeDtypeStruct((n, d), jnp.float32),
    )(s_prev, p0, p1, dg0, dg1)


def kernel(x, edge_index, W_self0, W_neigh0, b0, W_self1, W_neigh1, b1,
           W_self2, W_neigh2, b2):
    n, d = x.shape
    e = edge_index.shape[1]
    block = 1000 if n % 1000 == 0 else 8 * (n // 8)

    chunks_tot = -(-e // (CHUNK * NS * G)) * NS * G
    e_pad = chunks_tot * CHUNK
    cpw = chunks_tot // NW
    b_chunks = (chunks_tot // NS) // 5 // G * G
    a_chunks = chunks_tot // NS - b_chunks
    span = -(-(n + 1) // (NS * 8)) * 8
    n_pad = span * NS

    src = edge_index[0]
    dst = edge_index[1]
    trash = n + jnp.arange(e_pad - e, dtype=jnp.int32) % (n_pad - n)
    srcc = jnp.concatenate(
        [src, jnp.zeros((e_pad - e,), jnp.int32)]).reshape(chunks_tot, CHUNK)
    dstc = jnp.concatenate([dst, trash]).reshape(chunks_tot, CHUNK)

    deg2 = _make_deg_kernel(n_pad, d, cpw)(dstc)
    dg0 = deg2[0, :n, 0:1]
    dg1 = deg2[1, :n, 0:1]

    seg = _make_seg_kernel(n_pad, d, a_chunks, b_chunks)

    s0, m0 = _tc_first(_flt(x), _flt(W_self0), _flt(W_neigh0), _flt(b0), block)
    p = seg(m0, srcc, dstc)
    s1, m1 = _tc_mid(s0, p[0, :n], p[1, :n], dg0, dg1,
                     _flt(W_self1), _flt(W_neigh1), _flt(b1), block)
    p = seg(m1, srcc, dstc)
    s2, m2 = _tc_mid(s1, p[0, :n], p[1, :n], dg0, dg1,
                     _flt(W_self2), _flt(W_neigh2), _flt(b2), block)
    p = seg(m2, srcc, dstc)
    return _tc_last(s2, p[0, :n], p[1, :n], dg0, dg1, block)


# speedup vs baseline: 3.0774x; 2.7768x over previous
import functools

import jax
import jax.numpy as jnp
from jax import lax
from jax.experimental import pallas as pl
from jax.experimental.pallas import tpu as pltpu
from jax.experimental.pallas import tpu_sc as plsc

NC = 2
NS = 16
NW = NC * NS
CHUNK = 128
G = 40


def _flt(x):
    return x.astype(jnp.float32)


@functools.lru_cache(maxsize=None)
def _make_deg_kernel(n_pad, d, cpw):
    rows_per_sub = n_pad // NS
    nz_full, nz_rem = divmod(rows_per_sub, CHUNK)
    mesh = plsc.VectorSubcoreMesh(core_axis_name="c", subcore_axis_name="s")

    @functools.partial(
        pl.kernel,
        out_type=jax.ShapeDtypeStruct((NC, n_pad, d), jnp.float32),
        mesh=mesh,
        scratch_types=[
            pltpu.VMEM((cpw, CHUNK), jnp.int32),
            pltpu.VMEM((CHUNK, d), jnp.float32),
            pltpu.VMEM_SHARED((n_pad, d), jnp.float32),
        ],
    )
    def deg_kernel(dstc_hbm, out_hbm, dst_v, buf_v, deg_sh):
        c = lax.axis_index("c")
        s = lax.axis_index("s")
        wid = c * NS + s
        pltpu.sync_copy(dstc_hbm.at[pl.ds(wid * cpw, cpw)], dst_v)

        zero16 = jnp.zeros((16,), jnp.float32)

        def fill_zeros(i, _):
            for k in range(d // 16):
                buf_v[i, pl.ds(k * 16, 16)] = zero16
            return 0

        lax.fori_loop(0, CHUNK, fill_zeros, 0)
        base = s * rows_per_sub
        for k in range(nz_full):
            pltpu.sync_copy(buf_v, deg_sh.at[pl.ds(base + k * CHUNK, CHUNK)])
        if nz_rem:
            pltpu.sync_copy(buf_v.at[pl.ds(0, nz_rem)],
                            deg_sh.at[pl.ds(base + nz_full * CHUNK, nz_rem)])
        plsc.subcore_barrier()

        one16 = jnp.ones((16,), jnp.float32)

        def fill_ones(i, _):
            for k in range(d // 16):
                buf_v[i, pl.ds(k * 16, 16)] = one16
            return 0

        lax.fori_loop(0, CHUNK, fill_ones, 0)

        def scat(j, _):
            pltpu.sync_copy(buf_v, deg_sh.at[dst_v.at[j]], add=True)
            return 0

        lax.fori_loop(0, cpw, scat, 0)
        plsc.subcore_barrier()

        for k in range(nz_full):
            pltpu.sync_copy(deg_sh.at[pl.ds(base + k * CHUNK, CHUNK)], buf_v)
            pltpu.sync_copy(buf_v, out_hbm.at[c, pl.ds(base + k * CHUNK, CHUNK)])
        if nz_rem:
            pltpu.sync_copy(deg_sh.at[pl.ds(base + nz_full * CHUNK, nz_rem)],
                            buf_v.at[pl.ds(0, nz_rem)])
            pltpu.sync_copy(buf_v.at[pl.ds(0, nz_rem)],
                            out_hbm.at[c, pl.ds(base + nz_full * CHUNK, nz_rem)])

    return deg_kernel


@functools.lru_cache(maxsize=None)
def _make_seg_kernel(n_pad, d, a_chunks, b_chunks):
    rows_per_sub = n_pad // NS
    nz_full, nz_rem = divmod(rows_per_sub, CHUNK)
    ngroups_max = max(a_chunks, b_chunks) // G
    mesh = plsc.VectorSubcoreMesh(core_axis_name="c", subcore_axis_name="s")

    @functools.partial(
        pl.kernel,
        out_type=jax.ShapeDtypeStruct((NC, n_pad, d), jnp.float32),
        mesh=mesh,
        scratch_types=[
            pltpu.VMEM((G, CHUNK), jnp.int32),
            pltpu.VMEM((G, CHUNK), jnp.int32),
            pltpu.VMEM((CHUNK, d), jnp.float32),
            pltpu.VMEM((CHUNK, d), jnp.float32),
            pltpu.VMEM_SHARED((n_pad, d), jnp.float32),
            pltpu.SemaphoreType.DMA,
            pltpu.SemaphoreType.DMA,
            pltpu.SemaphoreType.DMA,
            pltpu.SemaphoreType.DMA,
        ],
    )
    def seg_kernel(m_hbm, srcc_hbm, dstc_hbm, out_hbm,
                   src_g, dst_g, rows0, rows1, agg_sh, gs0, gs1, ss0, ss1):
        c = lax.axis_index("c")
        s = lax.axis_index("s")
        e0 = jnp.where(c == 0, s * a_chunks, NS * a_chunks + s * b_chunks)
        ngroups = jnp.where(c == 0, a_chunks // G, b_chunks // G)

        zero16 = jnp.zeros((16,), jnp.float32)

        def fill_zeros(i, _):
            for k in range(d // 16):
                rows0[i, pl.ds(k * 16, 16)] = zero16
            return 0

        with jax.named_scope("segzero"):
            lax.fori_loop(0, CHUNK, fill_zeros, 0)
            zbase = s * rows_per_sub
            for k in range(nz_full):
                pltpu.sync_copy(rows0,
                                agg_sh.at[pl.ds(zbase + k * CHUNK, CHUNK)])
            if nz_rem:
                pltpu.sync_copy(rows0.at[pl.ds(0, nz_rem)],
                                agg_sh.at[pl.ds(zbase + nz_full * CHUNK,
                                                nz_rem)])
            plsc.subcore_barrier()

        def group(g, _):
            @pl.when(g < ngroups)
            def _():
                gbase = e0 + g * G
                pltpu.sync_copy(srcc_hbm.at[pl.ds(gbase, G)], src_g)
                pltpu.sync_copy(dstc_hbm.at[pl.ds(gbase, G)], dst_g)
                pltpu.async_copy(m_hbm.at[src_g.at[0]], rows0, gs0)

                def pair(t, _):
                    a = 2 * t
                    pltpu.make_async_copy(m_hbm.at[src_g.at[a]], rows0,
                                          gs0).wait()

                    @pl.when((g > 0) | (t > 0))
                    def _():
                        pltpu.make_async_copy(rows1, agg_sh.at[dst_g.at[a]],
                                              ss1).wait()

                    pltpu.async_copy(m_hbm.at[src_g.at[a + 1]], rows1, gs1)
                    pltpu.async_copy(rows0, agg_sh.at[dst_g.at[a]], ss0,
                                     add=True)
                    pltpu.make_async_copy(m_hbm.at[src_g.at[a + 1]], rows1,
                                          gs1).wait()
                    pltpu.make_async_copy(rows0, agg_sh.at[dst_g.at[a]],
                                          ss0).wait()

                    @pl.when(a + 2 < G)
                    def _():
                        pltpu.async_copy(m_hbm.at[src_g.at[a + 2]], rows0, gs0)

                    pltpu.async_copy(rows1, agg_sh.at[dst_g.at[a + 1]], ss1,
                                     add=True)
                    return 0

                lax.fori_loop(0, G // 2, pair, 0)
            return 0

        with jax.named_scope("segedges"):
            lax.fori_loop(0, ngroups_max, group, 0)
            @pl.when(ngroups > 0)
            def _():
                pltpu.make_async_copy(rows1, agg_sh.at[dst_g.at[G - 1]],
                                      ss1).wait()
            plsc.subcore_barrier()

        with jax.named_scope("segcopyout"):
            for k in range(nz_full):
                pltpu.sync_copy(agg_sh.at[pl.ds(zbase + k * CHUNK, CHUNK)],
                                rows0)
                pltpu.sync_copy(rows0,
                                out_hbm.at[c, pl.ds(zbase + k * CHUNK, CHUNK)])
            if nz_rem:
                pltpu.sync_copy(agg_sh.at[pl.ds(zbase + nz_full * CHUNK,
                                                nz_rem)],
                                rows0.at[pl.ds(0, nz_rem)])
                pltpu.sync_copy(rows0.at[pl.ds(0, nz_rem)],
                                out_hbm.at[c, pl.ds(zbase + nz_full * CHUNK,
                                                    nz_rem)])

    return seg_kernel


def _tc_first(x, w_self, w_neigh, b, block):
    n, d = x.shape
    h = w_self.shape[1]

    def body(x_ref, ws_ref, wn_ref, b_ref, s_ref, m_ref):
        xb = x_ref[...]
        s_ref[...] = jnp.dot(xb, ws_ref[...],
                             preferred_element_type=jnp.float32) + b_ref[...]
        m_ref[...] = jnp.dot(xb, wn_ref[...], preferred_element_type=jnp.float32)

    return pl.pallas_call(
        body,
        grid=(n // block,),
        in_specs=[
            pl.BlockSpec((block, d), lambda i: (i, 0)),
            pl.BlockSpec((d, h), lambda i: (0, 0)),
            pl.BlockSpec((d, h), lambda i: (0, 0)),
            pl.BlockSpec((1, h), lambda i: (0, 0)),
        ],
        out_specs=[
            pl.BlockSpec((block, h), lambda i: (i, 0)),
            pl.BlockSpec((block, h), lambda i: (i, 0)),
        ],
        out_shape=[
            jax.ShapeDtypeStruct((n, h), jnp.float32),
            jax.ShapeDtypeStruct((n, h), jnp.float32),
        ],
    )(x, w_self, w_neigh, b.reshape(1, h))


def _tc_mid(s_prev, p0, p1, dg0, dg1, w_self, w_neigh, b, block):
    n, d = s_prev.shape
    h = w_self.shape[1]

    def body(s_ref, p0_ref, p1_ref, dg0_ref, dg1_ref, ws_ref, wn_ref, b_ref,
             so_ref, mo_ref):
        inv = 1.0 / jnp.maximum(dg0_ref[...] + dg1_ref[...], 1.0)
        hb = jnp.maximum(s_ref[...] + (p0_ref[...] + p1_ref[...]) * inv, 0.0)
        so_ref[...] = jnp.dot(hb, ws_ref[...],
                              preferred_element_type=jnp.float32) + b_ref[...]
        mo_ref[...] = jnp.dot(hb, wn_ref[...], preferred_element_type=jnp.float32)

    return pl.pallas_call(
        body,
        grid=(n // block,),
        in_specs=[
            pl.BlockSpec((block, d), lambda i: (i, 0)),
            pl.BlockSpec((block, d), lambda i: (i, 0)),
            pl.BlockSpec((block, d), lambda i: (i, 0)),
            pl.BlockSpec((block, 1), lambda i: (i, 0)),
            pl.BlockSpec((block, 1), lambda i: (i, 0)),
            pl.BlockSpec((d, h), lambda i: (0, 0)),
            pl.BlockSpec((d, h), lambda i: (0, 0)),
            pl.BlockSpec((1, h), lambda i: (0, 0)),
        ],
        out_specs=[
            pl.BlockSpec((block, h), lambda i: (i, 0)),
            pl.BlockSpec((block, h), lambda i: (i, 0)),
        ],
        out_shape=[
            jax.ShapeDtypeStruct((n, h), jnp.float32),
            jax.ShapeDtypeStruct((n, h), jnp.float32),
        ],
    )(s_prev, p0, p1, dg0, dg1, w_self, w_neigh, b.reshape(1, h))


def _tc_last(s_prev, p0, p1, dg0, dg1, block):
    n, d = s_prev.shape

    def body(s_ref, p0_ref, p1_ref, dg0_ref, dg1_ref, o_ref):
        inv = 1.0 / jnp.maximum(dg0_ref[...] + dg1_ref[...], 1.0)
        o_ref[...] = s_ref[...] + (p0_ref[...] + p1_ref[...]) * inv

    return pl.pallas_call(
        body,
        grid=(n // block,),
        in_specs=[
            pl.BlockSpec((block, d), lambda i: (i, 0)),
            pl.BlockSpec((block, d), lambda i: (i, 0)),
            pl.BlockSpec((block, d), lambda i: (i, 0)),
            pl.BlockSpec((block, 1), lambda i: (i, 0)),
            pl.BlockSpec((block, 1), lambda i: (i, 0)),
        ],
        out_specs=pl.BlockSpec((block, d), lambda i: (i, 0)),
        out_shape=jax.ShapeDtypeStruct((n, d), jnp.float32),
    )(s_prev, p0, p1, dg0, dg1)


def kernel(x, edge_index, W_self0, W_neigh0, b0, W_self1, W_neigh1, b1,
           W_self2, W_neigh2, b2):
    n, d = x.shape
    e = edge_index.shape[1]
    block = 1000 if n % 1000 == 0 else 8 * (n // 8)

    chunks_tot = -(-e // (CHUNK * NS * G)) * NS * G
    e_pad = chunks_tot * CHUNK
    cpw = chunks_tot // NW
    a_chunks = chunks_tot // NW
    b_chunks = chunks_tot // NW
    span = -(-(n + 1) // (NS * 8)) * 8
    n_pad = span * NS

    src = edge_index[0]
    dst = edge_index[1]
    pad_iota = jnp.arange(e_pad - e, dtype=jnp.int32)
    srcc = jnp.concatenate([src, pad_iota % n]).reshape(chunks_tot, CHUNK)
    dstc = jnp.concatenate(
        [dst, n + pad_iota % (n_pad - n)]).reshape(chunks_tot, CHUNK)

    deg2 = _make_deg_kernel(n_pad, d, cpw)(dstc)
    dg0 = deg2[0, :n, 0:1]
    dg1 = deg2[1, :n, 0:1]

    seg = _make_seg_kernel(n_pad, d, a_chunks, b_chunks)

    s0, m0 = _tc_first(_flt(x), _flt(W_self0), _flt(W_neigh0), _flt(b0), block)
    p = seg(m0, srcc, dstc)
    s1, m1 = _tc_mid(s0, p[0, :n], p[1, :n], dg0, dg1,
                     _flt(W_self1), _flt(W_neigh1), _flt(b1), block)
    p = seg(m1, srcc, dstc)
    s2, m2 = _tc_mid(s1, p[0, :n], p[1, :n], dg0, dg1,
                     _flt(W_self2), _flt(W_neigh2), _flt(b2), block)
    p = seg(m2, srcc, dstc)
    return _tc_last(s2, p[0, :n], p[1, :n], dg0, dg1, block)


# speedup vs baseline: 3.5160x; 1.1426x over previous
import functools

import jax
import jax.numpy as jnp
from jax import lax
from jax.experimental import pallas as pl
from jax.experimental.pallas import tpu as pltpu
from jax.experimental.pallas import tpu_sc as plsc

NC = 2
NS = 16
NW = NC * NS
CHUNK = 128
G = 40


def _flt(x):
    return x.astype(jnp.float32)


@functools.lru_cache(maxsize=None)
def _make_deg_kernel(n_pad, d, cpw):
    rows_per_sub = n_pad // NS
    nz_full, nz_rem = divmod(rows_per_sub, CHUNK)
    mesh = plsc.VectorSubcoreMesh(core_axis_name="c", subcore_axis_name="s")

    @functools.partial(
        pl.kernel,
        out_type=jax.ShapeDtypeStruct((NC, n_pad, d), jnp.float32),
        mesh=mesh,
        scratch_types=[
            pltpu.VMEM((cpw, CHUNK), jnp.int32),
            pltpu.VMEM((CHUNK, d), jnp.float32),
            pltpu.VMEM_SHARED((n_pad, d), jnp.float32),
        ],
    )
    def deg_kernel(dstc_hbm, out_hbm, dst_v, buf_v, deg_sh):
        c = lax.axis_index("c")
        s = lax.axis_index("s")
        wid = c * NS + s
        pltpu.sync_copy(dstc_hbm.at[pl.ds(wid * cpw, cpw)], dst_v)

        zero16 = jnp.zeros((16,), jnp.float32)

        def fill_zeros(i, _):
            for k in range(d // 16):
                buf_v[i, pl.ds(k * 16, 16)] = zero16
            return 0

        lax.fori_loop(0, CHUNK, fill_zeros, 0)
        base = s * rows_per_sub
        for k in range(nz_full):
            pltpu.sync_copy(buf_v, deg_sh.at[pl.ds(base + k * CHUNK, CHUNK)])
        if nz_rem:
            pltpu.sync_copy(buf_v.at[pl.ds(0, nz_rem)],
                            deg_sh.at[pl.ds(base + nz_full * CHUNK, nz_rem)])
        plsc.subcore_barrier()

        one16 = jnp.ones((16,), jnp.float32)

        def fill_ones(i, _):
            for k in range(d // 16):
                buf_v[i, pl.ds(k * 16, 16)] = one16
            return 0

        lax.fori_loop(0, CHUNK, fill_ones, 0)

        def scat(j, _):
            pltpu.sync_copy(buf_v, deg_sh.at[dst_v.at[j]], add=True)
            return 0

        lax.fori_loop(0, cpw, scat, 0)
        plsc.subcore_barrier()

        for k in range(nz_full):
            pltpu.sync_copy(deg_sh.at[pl.ds(base + k * CHUNK, CHUNK)], buf_v)
            pltpu.sync_copy(buf_v, out_hbm.at[c, pl.ds(base + k * CHUNK, CHUNK)])
        if nz_rem:
            pltpu.sync_copy(deg_sh.at[pl.ds(base + nz_full * CHUNK, nz_rem)],
                            buf_v.at[pl.ds(0, nz_rem)])
            pltpu.sync_copy(buf_v.at[pl.ds(0, nz_rem)],
                            out_hbm.at[c, pl.ds(base + nz_full * CHUNK, nz_rem)])

    return deg_kernel


@functools.lru_cache(maxsize=None)
def _make_seg_kernel(n_pad, d, a_chunks, b_chunks):
    rows_per_sub = n_pad // NS
    nz_full, nz_rem = divmod(rows_per_sub, CHUNK)
    ngroups_max = max(a_chunks, b_chunks) // G
    mesh = plsc.VectorSubcoreMesh(core_axis_name="c", subcore_axis_name="s")

    @functools.partial(
        pl.kernel,
        out_type=jax.ShapeDtypeStruct((NC, n_pad, d), jnp.float32),
        mesh=mesh,
        scratch_types=[
            pltpu.VMEM((G, CHUNK), jnp.int32),
            pltpu.VMEM((G, CHUNK), jnp.int32),
            pltpu.VMEM((CHUNK, d), jnp.float32),
            pltpu.VMEM((CHUNK, d), jnp.float32),
            pltpu.VMEM_SHARED((n_pad, d), jnp.float32),
            pltpu.SemaphoreType.DMA,
            pltpu.SemaphoreType.DMA,
            pltpu.SemaphoreType.DMA,
            pltpu.SemaphoreType.DMA,
        ],
    )
    def seg_kernel(m_hbm, srcc_hbm, dstc_hbm, out_hbm,
                   src_g, dst_g, rows0, rows1, agg_sh, gs0, gs1, ss0, ss1):
        c = lax.axis_index("c")
        s = lax.axis_index("s")
        e0 = jnp.where(c == 0, s * a_chunks, NS * a_chunks + s * b_chunks)
        ngroups = jnp.where(c == 0, a_chunks // G, b_chunks // G)

        zero16 = jnp.zeros((16,), jnp.float32)

        def fill_zeros(i, _):
            for k in range(d // 16):
                rows0[i, pl.ds(k * 16, 16)] = zero16
            return 0

        with jax.named_scope("segzero"):
            lax.fori_loop(0, CHUNK, fill_zeros, 0)
            zbase = s * rows_per_sub
            for k in range(nz_full):
                pltpu.sync_copy(rows0,
                                agg_sh.at[pl.ds(zbase + k * CHUNK, CHUNK)])
            if nz_rem:
                pltpu.sync_copy(rows0.at[pl.ds(0, nz_rem)],
                                agg_sh.at[pl.ds(zbase + nz_full * CHUNK,
                                                nz_rem)])
            plsc.subcore_barrier()

        def group(g, _):
            @pl.when(g < ngroups)
            def _():
                gbase = e0 + g * G
                pltpu.sync_copy(srcc_hbm.at[pl.ds(gbase, G)], src_g)
                pltpu.sync_copy(dstc_hbm.at[pl.ds(gbase, G)], dst_g)
                pltpu.async_copy(m_hbm.at[src_g.at[0]], rows0, gs0)

                def pair(t, _):
                    a = 2 * t
                    pltpu.make_async_copy(m_hbm.at[src_g.at[a]], rows0,
                                          gs0).wait()

                    @pl.when((g > 0) | (t > 0))
                    def _():
                        pltpu.make_async_copy(rows1, agg_sh.at[dst_g.at[a]],
                                              ss1).wait()

                    pltpu.async_copy(m_hbm.at[src_g.at[a + 1]], rows1, gs1)
                    pltpu.async_copy(rows0, agg_sh.at[dst_g.at[a]], ss0,
                                     add=True)
                    pltpu.make_async_copy(m_hbm.at[src_g.at[a + 1]], rows1,
                                          gs1).wait()
                    pltpu.make_async_copy(rows0, agg_sh.at[dst_g.at[a]],
                                          ss0).wait()

                    @pl.when(a + 2 < G)
                    def _():
                        pltpu.async_copy(m_hbm.at[src_g.at[a + 2]], rows0, gs0)

                    pltpu.async_copy(rows1, agg_sh.at[dst_g.at[a + 1]], ss1,
                                     add=True)
                    return 0

                lax.fori_loop(0, G // 2, pair, 0)
            return 0

        with jax.named_scope("segedges"):
            lax.fori_loop(0, ngroups_max, group, 0)
            @pl.when(ngroups > 0)
            def _():
                pltpu.make_async_copy(rows1, agg_sh.at[dst_g.at[G - 1]],
                                      ss1).wait()
            plsc.subcore_barrier()

        with jax.named_scope("segcopyout"):
            for k in range(nz_full):
                pltpu.sync_copy(agg_sh.at[pl.ds(zbase + k * CHUNK, CHUNK)],
                                rows0)
                pltpu.sync_copy(rows0,
                                out_hbm.at[c, pl.ds(zbase + k * CHUNK, CHUNK)])
            if nz_rem:
                pltpu.sync_copy(agg_sh.at[pl.ds(zbase + nz_full * CHUNK,
                                                nz_rem)],
                                rows0.at[pl.ds(0, nz_rem)])
                pltpu.sync_copy(rows0.at[pl.ds(0, nz_rem)],
                                out_hbm.at[c, pl.ds(zbase + nz_full * CHUNK,
                                                    nz_rem)])

    return seg_kernel


def _tc_first(x, w_self, w_neigh, b, block):
    n, d = x.shape
    h = w_self.shape[1]

    def body(x_ref, ws_ref, wn_ref, b_ref, s_ref, m_ref):
        xb = x_ref[...]
        s_ref[...] = jnp.dot(xb, ws_ref[...],
                             preferred_element_type=jnp.float32) + b_ref[...]
        m_ref[...] = jnp.dot(xb, wn_ref[...], preferred_element_type=jnp.float32)

    return pl.pallas_call(
        body,
        grid=(n // block,),
        in_specs=[
            pl.BlockSpec((block, d), lambda i: (i, 0)),
            pl.BlockSpec((d, h), lambda i: (0, 0)),
            pl.BlockSpec((d, h), lambda i: (0, 0)),
            pl.BlockSpec((1, h), lambda i: (0, 0)),
        ],
        out_specs=[
            pl.BlockSpec((block, h), lambda i: (i, 0)),
            pl.BlockSpec((block, h), lambda i: (i, 0)),
        ],
        out_shape=[
            jax.ShapeDtypeStruct((n, h), jnp.float32),
            jax.ShapeDtypeStruct((n, h), jnp.float32),
        ],
    )(x, w_self, w_neigh, b.reshape(1, h))


def _tc_mid(s_prev, p, deg2, w_self, w_neigh, b, block):
    n, d = s_prev.shape
    h = w_self.shape[1]
    dd = deg2.shape[2]

    def body(s_ref, p0_ref, p1_ref, dg0_ref, dg1_ref, ws_ref, wn_ref, b_ref,
             so_ref, mo_ref):
        inv = 1.0 / jnp.maximum(dg0_ref[0, :, 0:1] + dg1_ref[0, :, 0:1], 1.0)
        hb = jnp.maximum(s_ref[...] + (p0_ref[0] + p1_ref[0]) * inv, 0.0)
        so_ref[...] = jnp.dot(hb, ws_ref[...],
                              preferred_element_type=jnp.float32) + b_ref[...]
        mo_ref[...] = jnp.dot(hb, wn_ref[...], preferred_element_type=jnp.float32)

    return pl.pallas_call(
        body,
        grid=(n // block,),
        in_specs=[
            pl.BlockSpec((block, d), lambda i: (i, 0)),
            pl.BlockSpec((1, block, d), lambda i: (0, i, 0)),
            pl.BlockSpec((1, block, d), lambda i: (1, i, 0)),
            pl.BlockSpec((1, block, dd), lambda i: (0, i, 0)),
            pl.BlockSpec((1, block, dd), lambda i: (1, i, 0)),
            pl.BlockSpec((d, h), lambda i: (0, 0)),
            pl.BlockSpec((d, h), lambda i: (0, 0)),
            pl.BlockSpec((1, h), lambda i: (0, 0)),
        ],
        out_specs=[
            pl.BlockSpec((block, h), lambda i: (i, 0)),
            pl.BlockSpec((block, h), lambda i: (i, 0)),
        ],
        out_shape=[
            jax.ShapeDtypeStruct((n, h), jnp.float32),
            jax.ShapeDtypeStruct((n, h), jnp.float32),
        ],
    )(s_prev, p, p, deg2, deg2, w_self, w_neigh, b.reshape(1, h))


def _tc_last(s_prev, p, deg2, block):
    n, d = s_prev.shape
    dd = deg2.shape[2]

    def body(s_ref, p0_ref, p1_ref, dg0_ref, dg1_ref, o_ref):
        inv = 1.0 / jnp.maximum(dg0_ref[0, :, 0:1] + dg1_ref[0, :, 0:1], 1.0)
        o_ref[...] = s_ref[...] + (p0_ref[0] + p1_ref[0]) * inv

    return pl.pallas_call(
        body,
        grid=(n // block,),
        in_specs=[
            pl.BlockSpec((block, d), lambda i: (i, 0)),
            pl.BlockSpec((1, block, d), lambda i: (0, i, 0)),
            pl.BlockSpec((1, block, d), lambda i: (1, i, 0)),
            pl.BlockSpec((1, block, dd), lambda i: (0, i, 0)),
            pl.BlockSpec((1, block, dd), lambda i: (1, i, 0)),
        ],
        out_specs=pl.BlockSpec((block, d), lambda i: (i, 0)),
        out_shape=jax.ShapeDtypeStruct((n, d), jnp.float32),
    )(s_prev, p, p, deg2, deg2)


def kernel(x, edge_index, W_self0, W_neigh0, b0, W_self1, W_neigh1, b1,
           W_self2, W_neigh2, b2):
    n, d = x.shape
    e = edge_index.shape[1]
    block = 1000 if n % 1000 == 0 else 8 * (n // 8)

    chunks_tot = -(-e // (CHUNK * NS * G)) * NS * G
    e_pad = chunks_tot * CHUNK
    cpw = chunks_tot // NW
    a_chunks = chunks_tot // NW
    b_chunks = chunks_tot // NW
    span = -(-(n + 1) // (NS * 8)) * 8
    n_pad = span * NS

    src = edge_index[0]
    dst = edge_index[1]
    pad_iota = jnp.arange(e_pad - e, dtype=jnp.int32)
    srcc = jnp.concatenate([src, pad_iota % n]).reshape(chunks_tot, CHUNK)
    dstc = jnp.concatenate(
        [dst, n + pad_iota % (n_pad - n)]).reshape(chunks_tot, CHUNK)

    deg2 = _make_deg_kernel(n_pad, 16, cpw)(dstc)

    seg = _make_seg_kernel(n_pad, d, a_chunks, b_chunks)

    s0, m0 = _tc_first(_flt(x), _flt(W_self0), _flt(W_neigh0), _flt(b0), block)
    p = seg(m0, srcc, dstc)
    s1, m1 = _tc_mid(s0, p, deg2,
                     _flt(W_self1), _flt(W_neigh1), _flt(b1), block)
    p = seg(m1, srcc, dstc)
    s2, m2 = _tc_mid(s1, p, deg2,
                     _flt(W_self2), _flt(W_neigh2), _flt(b2), block)
    p = seg(m2, srcc, dstc)
    return _tc_last(s2, p, deg2, block)
